# baseline (speedup 1.0000x reference)
import jax
import jax.numpy as jnp
from jax import lax
from jax.experimental import pallas as pl
from jax.experimental.pallas import tpu as pltpu
from jax.experimental.pallas import tpu_sc as plsc

N = 10000
E = 320000
D = 128
W = 144
NC = 2
NS = 16
NW = NC * NS
EPW = E // NW
B = 50
NB = EPW // B
CH = 8
NCH = NB // CH
NPT = N // NS

_f32 = jnp.float32
_i32 = jnp.int32

_SC_PARAMS = pltpu.CompilerParams(use_tc_tiling_on_sc=False,
                                  needs_layout_passes=False)
_MESH = dict(core_axis_name="c", subcore_axis_name="s")


def _sc_logits_body(es_hbm, ed_hbm, m_hbm, src_hbm, dst_hbm, w_out,
                    es_v, ed_v, m_v, src_v, dst_v, w_v):
    c = lax.axis_index("c")
    s = lax.axis_index("s")
    wid = c * NS + s

    pltpu.sync_copy(es_hbm, es_v)
    pltpu.sync_copy(ed_hbm, ed_v)
    pltpu.sync_copy(m_hbm, m_v)
    pltpu.sync_copy(src_hbm.at[wid], src_v)
    pltpu.sync_copy(dst_hbm.at[wid], dst_v)

    mval = m_v[pl.ds(0, 16)][0]

    def _pa(i, carry):
        sl = pl.ds(i * 16, 16)
        si = src_v[sl]
        di = dst_v[sl]
        ev = plsc.load_gather(es_v, [si])
        dv = plsc.load_gather(ed_v, [di])
        t = ev + dv
        e = jnp.where(t >= 0.0, t, t * _f32(0.2))
        w_v[sl] = jnp.exp(e - mval)
        return carry

    lax.fori_loop(0, EPW // 16, _pa, 0)
    pltpu.sync_copy(w_v, w_out.at[wid])


_sc_logits = pl.kernel(
    _sc_logits_body,
    out_type=jax.ShapeDtypeStruct((NW, EPW), _f32),
    mesh=plsc.VectorSubcoreMesh(**_MESH),
    compiler_params=_SC_PARAMS,
    scratch_types=[
        pltpu.VMEM((N,), _f32),
        pltpu.VMEM((N,), _f32),
        pltpu.VMEM((16,), _f32),
        pltpu.VMEM((EPW,), _i32),
        pltpu.VMEM((EPW,), _i32),
        pltpu.VMEM((EPW,), _f32),
    ],
)


def _sc_agg_body(h_hbm, src_hbm, dst_hbm, w_hbm,
                 acc_out,
                 srcR, dstR, wR, gbuf, sbuf,
                 acc_sh,
                 gsem0, gsem1, ssem0, ssem1, rsem):
    c = lax.axis_index("c")
    s = lax.axis_index("s")
    wid = c * NS + s
    gsems = (gsem0, gsem1)
    ssems = (ssem0, ssem1)
    lane0 = lax.iota(_i32, 16) == 0
    zero16 = jnp.zeros((16,), _f32)

    def _zrow(r, carry):
        for p in range(2):
            for ch in range(W // 16):
                sbuf[p, r, pl.ds(ch * 16, 16)] = zero16
        return carry

    lax.fori_loop(0, B, _zrow, 0)
    base = s * NPT
    nfull = NPT // B
    for j in range(nfull):
        pltpu.sync_copy(sbuf.at[0, pl.ds(0, B), :],
                        acc_sh.at[pl.ds(base + j * B, B), :])
    rem = NPT - nfull * B
    if rem:
        pltpu.sync_copy(sbuf.at[0, pl.ds(0, rem), :],
                        acc_sh.at[pl.ds(base + nfull * B, rem), :])
    plsc.subcore_barrier()

    def _ring_issue(cb2):
        slot2 = lax.rem(cb2, 2)
        sl2 = pl.ds(cb2 * CH, CH)
        pltpu.async_copy(src_hbm.at[wid, sl2], srcR.at[slot2], rsem)
        pltpu.async_copy(dst_hbm.at[wid, sl2], dstR.at[slot2], rsem)
        pltpu.async_copy(w_hbm.at[wid, sl2], wR.at[slot2], rsem)

    def _ring_wait():
        sl0 = pl.ds(0, CH)
        pltpu.make_async_copy(src_hbm.at[wid, sl0], srcR.at[0], rsem).wait()
        pltpu.make_async_copy(dst_hbm.at[wid, sl0], dstR.at[0], rsem).wait()
        pltpu.make_async_copy(w_hbm.at[wid, sl0], wR.at[0], rsem).wait()

    def _gissue(slot, j, p):
        pltpu.async_copy(h_hbm.at[srcR.at[slot, j]], gbuf.at[p], gsems[p])

    def _gwait(slot, j, p):
        pltpu.make_async_copy(h_hbm.at[srcR.at[slot, j]],
                              gbuf.at[p], gsems[p]).wait()

    def _sissue(slot, j, p):
        pltpu.async_copy(sbuf.at[p], acc_sh.at[dstR.at[slot, j]],
                         ssems[p], add=True)

    def _swait(p):
        pltpu.make_async_copy(sbuf.at[p], acc_sh.at[dstR.at[0, 0]],
                              ssems[p]).wait()

    iota16 = lax.iota(_i32, 16)
    colD = jnp.full((16,), D, _i32)

    def _scale_batch(p, slot, j):
        q16s = [q * 16 for q in range(B // 16)]
        if B % 16:
            q16s.append(B - 16)
        for gi, q16 in enumerate(q16s):
            wv = wR[slot, j, pl.ds(q16, 16)]
            lo = 0 if gi < len(q16s) - 1 or not B % 16 else 16 - (B % 16)
            for j2 in range(lo, 16):
                r = q16 + j2
                wsc = wv[j2]
                loads = [gbuf[p, r, pl.ds(ch * 16, 16)]
                         for ch in range(D // 16)]
                for ch in range(D // 16):
                    sbuf[p, r, pl.ds(ch * 16, 16)] = loads[ch] * wsc
            plsc.store_scatter(
                sbuf,
                [jnp.full((16,), p, _i32), q16 + iota16, colD],
                wv)

    _ring_issue(0)
    _ring_wait()
    _gissue(0, 0, 0)
    _gissue(0, 1, 1)

    def _chunk(cb, carry):
        slot = lax.rem(cb, 2)
        nslot = 1 - slot

        @pl.when(cb + 1 < NCH)
        def _():
            _ring_issue(cb + 1)

        def _pair(jj, c2):
            @pl.when(jnp.logical_and(jj == 1, cb + 1 < NCH))
            def _():
                _ring_wait()

            for p in range(2):
                j = 2 * jj + p
                g = cb * CH + j
                _gwait(slot, j, p)

                @pl.when(g >= 2)
                def _():
                    _swait(p)

                _scale_batch(p, slot, j)
                _sissue(slot, j, p)

                @pl.when(jnp.logical_and(j + 2 < CH, g + 2 < NB))
                def _():
                    _gissue(slot, j + 2, p)

                @pl.when(jnp.logical_and(j + 2 >= CH, g + 2 < NB))
                def _():
                    _gissue(nslot, j + 2 - CH, p)
            return c2

        lax.fori_loop(0, CH // 2, _pair, 0)
        return carry

    lax.fori_loop(0, NCH, _chunk, 0)
    _swait(0)
    _swait(1)
    plsc.subcore_barrier()

    pltpu.sync_copy(acc_sh.at[pl.ds(base, NPT), :],
                    acc_out.at[c, pl.ds(base, NPT), :])


_sc_agg = pl.kernel(
    _sc_agg_body,
    out_type=jax.ShapeDtypeStruct((NC, N, W), _f32),
    mesh=plsc.VectorSubcoreMesh(**_MESH),
    compiler_params=_SC_PARAMS,
    scratch_types=(
        [pltpu.VMEM((2, CH, B), _i32),
         pltpu.VMEM((2, CH, B), _i32),
         pltpu.VMEM((2, CH, B), _f32),
         pltpu.VMEM((2, B, D), _f32),
         pltpu.VMEM((2, B, W), _f32)]
        + [pltpu.VMEM_SHARED((N, W), _f32)]
        + [pltpu.SemaphoreType.DMA for _ in range(5)]
    ),
)


def _dense_tail(h, asrc_ref, adst_ref, h_ref, es_ref, ed_ref, m_ref):
    h_ref[...] = h
    es = jnp.dot(h, asrc_ref[...], preferred_element_type=_f32)
    ed = jnp.dot(h, adst_ref[...], preferred_element_type=_f32)
    es_ref[...] = es
    ed_ref[...] = ed
    mm = jnp.max(es) + jnp.max(ed)
    mb = jnp.where(mm >= 0.0, mm, mm * 0.2)
    m_ref[...] = jnp.full((1, 16), mb, _f32)


def _tc1_body(x_ref, w_ref, asrc_ref, adst_ref, h_ref, es_ref, ed_ref, m_ref):
    h = jnp.dot(x_ref[...], w_ref[...], preferred_element_type=_f32)
    _dense_tail(h, asrc_ref, adst_ref, h_ref, es_ref, ed_ref, m_ref)


def _combine(acc_ref, b_ref):
    a = acc_ref[0, :, :D] + acc_ref[1, :, :D]
    den = acc_ref[0, :, D:D + 1] + acc_ref[1, :, D:D + 1]
    den = jnp.where(den == 0.0, _f32(1.0), den)
    return jnp.maximum(a / den + b_ref[...], 0.0)


def _tc2_body(acc_ref, b_ref, w_ref, asrc_ref, adst_ref,
              h_ref, es_ref, ed_ref, m_ref):
    emb = _combine(acc_ref, b_ref)
    h = jnp.dot(emb, w_ref[...], preferred_element_type=_f32)
    _dense_tail(h, asrc_ref, adst_ref, h_ref, es_ref, ed_ref, m_ref)


def _tc3_body(acc_ref, b_ref, wo_ref, bo_ref, o_ref):
    emb = _combine(acc_ref, b_ref)
    o_ref[...] = jnp.dot(emb, wo_ref[...], preferred_element_type=_f32) + bo_ref[...]


_dense_out = (jax.ShapeDtypeStruct((N, D), _f32),
              jax.ShapeDtypeStruct((N, 1), _f32),
              jax.ShapeDtypeStruct((N, 1), _f32),
              jax.ShapeDtypeStruct((1, 16), _f32))

_tc1 = pl.pallas_call(_tc1_body, out_shape=_dense_out)
_tc2 = pl.pallas_call(_tc2_body, out_shape=_dense_out)
_tc3 = pl.pallas_call(_tc3_body, out_shape=jax.ShapeDtypeStruct((N, 1), _f32))


def kernel(x, edge_index, W1, a1_src, a1_dst, b1, W2, a2_src, a2_dst, b2, Wo, bo):
    srcf = edge_index[0].reshape(NW, EPW)
    dstf = edge_index[1].reshape(NW, EPW)
    src3 = edge_index[0].reshape(NW, NB, B)
    dst3 = edge_index[1].reshape(NW, NB, B)

    def layer(h, es, ed, m):
        w = _sc_logits(es.reshape(N), ed.reshape(N), m.reshape(16), srcf, dstf)
        return _sc_agg(h, src3, dst3, w.reshape(NW, NB, B))

    h1, es1, ed1, m1 = _tc1(x, W1, a1_src.reshape(D, 1), a1_dst.reshape(D, 1))
    acc1 = layer(h1, es1, ed1, m1)
    h2, es2, ed2, m2 = _tc2(acc1, b1.reshape(1, D), W2,
                            a2_src.reshape(D, 1), a2_dst.reshape(D, 1))
    acc2 = layer(h2, es2, ed2, m2)
    return _tc3(acc2, b2.reshape(1, D), Wo, bo.reshape(1, 1))


# speedup vs baseline: 1.0160x; 1.0160x over previous
import jax
import jax.numpy as jnp
from jax import lax
from jax.experimental import pallas as pl
from jax.experimental.pallas import tpu as pltpu
from jax.experimental.pallas import tpu_sc as plsc

N = 10000
E = 320000
D = 128
W = 144
NC = 2
NS = 16
NW = NC * NS
EPW = E // NW
B = 50
NB = EPW // B
CH = 8
NCH = NB // CH
NPT = N // NS

_f32 = jnp.float32
_i32 = jnp.int32

_SC_PARAMS = pltpu.CompilerParams(use_tc_tiling_on_sc=False,
                                  needs_layout_passes=False)


def _sc_agg_body(h_hbm, ed_hbm, m_hbm, src_hbm, dst_hbm,
                 acc_out,
                 srcR, dstR, gbuf, ebuf, sbuf, m_v,
                 acc_sh,
                 gsem0, gsem1, esem0, esem1, ssem0, ssem1, rsem):
    c = lax.axis_index("c")
    s = lax.axis_index("s")
    wid = c * NS + s
    gsems = (gsem0, gsem1)
    esems = (esem0, esem1)
    ssems = (ssem0, ssem1)
    zero16 = jnp.zeros((16,), _f32)
    iota16 = lax.iota(_i32, 16)
    colD = jnp.full((16,), D, _i32)
    col0 = jnp.full((16,), 0, _i32)
    psplat = (jnp.full((16,), 0, _i32), jnp.full((16,), 1, _i32))

    pltpu.sync_copy(m_hbm, m_v)
    mval = m_v[pl.ds(0, 16)][0]

    def _zrow(r, carry):
        for p in range(2):
            for ch in range(W // 16):
                sbuf[p, r, pl.ds(ch * 16, 16)] = zero16
        return carry

    lax.fori_loop(0, B, _zrow, 0)
    base = s * NPT
    nfull = NPT // B
    for j in range(nfull):
        pltpu.sync_copy(sbuf.at[0, pl.ds(0, B), :],
                        acc_sh.at[pl.ds(base + j * B, B), :])
    rem = NPT - nfull * B
    if rem:
        pltpu.sync_copy(sbuf.at[0, pl.ds(0, rem), :],
                        acc_sh.at[pl.ds(base + nfull * B, rem), :])
    plsc.subcore_barrier()

    def _ring_issue(cb2):
        slot2 = lax.rem(cb2, 2)
        sl2 = pl.ds(cb2 * CH, CH)
        pltpu.async_copy(src_hbm.at[wid, sl2], srcR.at[slot2], rsem)
        pltpu.async_copy(dst_hbm.at[wid, sl2], dstR.at[slot2], rsem)

    def _ring_wait():
        sl0 = pl.ds(0, CH)
        pltpu.make_async_copy(src_hbm.at[wid, sl0], srcR.at[0], rsem).wait()
        pltpu.make_async_copy(dst_hbm.at[wid, sl0], dstR.at[0], rsem).wait()

    def _gissue(slot, j, p):
        pltpu.async_copy(h_hbm.at[srcR.at[slot, j]], gbuf.at[p], gsems[p])
        pltpu.async_copy(ed_hbm.at[dstR.at[slot, j]], ebuf.at[p], esems[p])

    def _gwait(slot, j, p):
        pltpu.make_async_copy(h_hbm.at[srcR.at[slot, j]],
                              gbuf.at[p], gsems[p]).wait()
        pltpu.make_async_copy(ed_hbm.at[dstR.at[slot, j]],
                              ebuf.at[p], esems[p]).wait()

    def _sissue(slot, j, p):
        pltpu.async_copy(sbuf.at[p], acc_sh.at[dstR.at[slot, j]],
                         ssems[p], add=True)

    def _swait(p):
        pltpu.make_async_copy(sbuf.at[p], acc_sh.at[dstR.at[0, 0]],
                              ssems[p]).wait()

    def _scale_batch(p):
        q16s = [q * 16 for q in range(B // 16)]
        if B % 16:
            q16s.append(B - 16)
        for gi, q16 in enumerate(q16s):
            rows = q16 + iota16
            esv = plsc.load_gather(gbuf, [psplat[p], rows, colD])
            edv = plsc.load_gather(ebuf, [psplat[p], rows, col0])
            t = esv + edv
            e = jnp.where(t >= 0.0, t, t * _f32(0.2))
            wv = jnp.exp(e - mval)
            lo = 0 if gi < len(q16s) - 1 or not B % 16 else 16 - (B % 16)
            for j2 in range(lo, 16):
                r = q16 + j2
                wsc = wv[j2]
                loads = [gbuf[p, r, pl.ds(ch * 16, 16)]
                         for ch in range(D // 16)]
                for ch in range(D // 16):
                    sbuf[p, r, pl.ds(ch * 16, 16)] = loads[ch] * wsc
            plsc.store_scatter(sbuf, [psplat[p], rows, colD], wv)

    _ring_issue(0)
    _ring_wait()
    _gissue(0, 0, 0)
    _gissue(0, 1, 1)

    def _chunk(cb, carry):
        slot = lax.rem(cb, 2)
        nslot = 1 - slot

        @pl.when(cb + 1 < NCH)
        def _():
            _ring_issue(cb + 1)

        def _pair(jj, c2):
            @pl.when(jnp.logical_and(jj == 1, cb + 1 < NCH))
            def _():
                _ring_wait()

            for p in range(2):
                j = 2 * jj + p
                g = cb * CH + j
                _gwait(slot, j, p)

                @pl.when(g >= 2)
                def _():
                    _swait(p)

                _scale_batch(p)
                _sissue(slot, j, p)

                @pl.when(jnp.logical_and(j + 2 < CH, g + 2 < NB))
                def _():
                    _gissue(slot, j + 2, p)

                @pl.when(jnp.logical_and(j + 2 >= CH, g + 2 < NB))
                def _():
                    _gissue(nslot, j + 2 - CH, p)
            return c2

        lax.fori_loop(0, CH // 2, _pair, 0)
        return carry

    lax.fori_loop(0, NCH, _chunk, 0)
    _swait(0)
    _swait(1)
    plsc.subcore_barrier()

    pltpu.sync_copy(acc_sh.at[pl.ds(base, NPT), :],
                    acc_out.at[c, pl.ds(base, NPT), :])


_sc_agg = pl.kernel(
    _sc_agg_body,
    out_type=jax.ShapeDtypeStruct((NC, N, W), _f32),
    mesh=plsc.VectorSubcoreMesh(core_axis_name="c", subcore_axis_name="s"),
    compiler_params=_SC_PARAMS,
    scratch_types=(
        [pltpu.VMEM((2, CH, B), _i32),
         pltpu.VMEM((2, CH, B), _i32),
         pltpu.VMEM((2, B, W), _f32),
         pltpu.VMEM((2, B, 16), _f32),
         pltpu.VMEM((2, B, W), _f32),
         pltpu.VMEM((16,), _f32)]
        + [pltpu.VMEM_SHARED((N, W), _f32)]
        + [pltpu.SemaphoreType.DMA for _ in range(7)]
    ),
)


def _dense_tail(h, asrc_ref, adst_ref, hp_ref, ed_ref, m_ref):
    es = jnp.dot(h, asrc_ref[...], preferred_element_type=_f32)
    ed = jnp.dot(h, adst_ref[...], preferred_element_type=_f32)
    hp_ref[...] = jnp.concatenate(
        [h, es, jnp.zeros((N, W - D - 1), _f32)], axis=1)
    ed_ref[...] = jnp.concatenate([ed, jnp.zeros((N, 15), _f32)], axis=1)
    mm = jnp.max(es) + jnp.max(ed)
    mb = jnp.where(mm >= 0.0, mm, mm * 0.2)
    m_ref[...] = jnp.full((1, 16), mb, _f32)


def _tc1_body(x_ref, w_ref, asrc_ref, adst_ref, hp_ref, ed_ref, m_ref):
    h = jnp.dot(x_ref[...], w_ref[...], preferred_element_type=_f32)
    _dense_tail(h, asrc_ref, adst_ref, hp_ref, ed_ref, m_ref)


def _combine(acc_ref, b_ref):
    a = acc_ref[0, :, :D] + acc_ref[1, :, :D]
    den = acc_ref[0, :, D:D + 1] + acc_ref[1, :, D:D + 1]
    den = jnp.where(den == 0.0, _f32(1.0), den)
    return jnp.maximum(a / den + b_ref[...], 0.0)


def _tc2_body(acc_ref, b_ref, w_ref, asrc_ref, adst_ref,
              hp_ref, ed_ref, m_ref):
    emb = _combine(acc_ref, b_ref)
    h = jnp.dot(emb, w_ref[...], preferred_element_type=_f32)
    _dense_tail(h, asrc_ref, adst_ref, hp_ref, ed_ref, m_ref)


def _tc3_body(acc_ref, b_ref, wo_ref, bo_ref, o_ref):
    emb = _combine(acc_ref, b_ref)
    o_ref[...] = jnp.dot(emb, wo_ref[...], preferred_element_type=_f32) + bo_ref[...]


_dense_out = (jax.ShapeDtypeStruct((N, W), _f32),
              jax.ShapeDtypeStruct((N, 16), _f32),
              jax.ShapeDtypeStruct((1, 16), _f32))

_tc1 = pl.pallas_call(_tc1_body, out_shape=_dense_out)
_tc2 = pl.pallas_call(_tc2_body, out_shape=_dense_out)
_tc3 = pl.pallas_call(_tc3_body, out_shape=jax.ShapeDtypeStruct((N, 1), _f32))


def kernel(x, edge_index, W1, a1_src, a1_dst, b1, W2, a2_src, a2_dst, b2, Wo, bo):
    src3 = edge_index[0].reshape(NW, NB, B)
    dst3 = edge_index[1].reshape(NW, NB, B)

    hp1, ed1, m1 = _tc1(x, W1, a1_src.reshape(D, 1), a1_dst.reshape(D, 1))
    acc1 = _sc_agg(hp1, ed1, m1.reshape(16), src3, dst3)
    hp2, ed2, m2 = _tc2(acc1, b1.reshape(1, D), W2,
                        a2_src.reshape(D, 1), a2_dst.reshape(D, 1))
    acc2 = _sc_agg(hp2, ed2, m2.reshape(16), src3, dst3)
    return _tc3(acc2, b2.reshape(1, D), Wo, bo.reshape(1, 1))


# speedup vs baseline: 1.0196x; 1.0036x over previous
import jax
import jax.numpy as jnp
from jax import lax
from jax.experimental import pallas as pl
from jax.experimental.pallas import tpu as pltpu
from jax.experimental.pallas import tpu_sc as plsc

N = 10000
E = 320000
D = 128
W = 144
NC = 2
NS = 16
NW = NC * NS
EPW = E // NW
B = 50
NB = EPW // B
CH = 8
NCH = NB // CH
NPT = N // NS

_f32 = jnp.float32
_i32 = jnp.int32

_SC_PARAMS = pltpu.CompilerParams(use_tc_tiling_on_sc=False,
                                  needs_layout_passes=False)


def _sc_agg_body(h_hbm, ed_hbm, m_hbm, src_hbm, dst_hbm,
                 acc_out,
                 srcR, dstR, gbuf, ebuf, sbuf, m_v,
                 acc_sh,
                 gsem0, gsem1, esem0, esem1, ssem0, ssem1, rsem):
    c = lax.axis_index("c")
    s = lax.axis_index("s")
    wid = c * NS + s
    gsems = (gsem0, gsem1)
    esems = (esem0, esem1)
    ssems = (ssem0, ssem1)
    zero16 = jnp.zeros((16,), _f32)
    iota16 = lax.iota(_i32, 16)
    colD = jnp.full((16,), D, _i32)
    col0 = jnp.full((16,), 0, _i32)
    psplat = (jnp.full((16,), 0, _i32), jnp.full((16,), 1, _i32))

    def _ring_issue(cb2):
        slot2 = lax.rem(cb2, 2)
        sl2 = pl.ds(cb2 * CH, CH)
        pltpu.async_copy(src_hbm.at[wid, sl2], srcR.at[slot2], rsem)
        pltpu.async_copy(dst_hbm.at[wid, sl2], dstR.at[slot2], rsem)

    def _ring_wait():
        sl0 = pl.ds(0, CH)
        pltpu.make_async_copy(src_hbm.at[wid, sl0], srcR.at[0], rsem).wait()
        pltpu.make_async_copy(dst_hbm.at[wid, sl0], dstR.at[0], rsem).wait()

    def _gissue(slot, j, p):
        pltpu.async_copy(h_hbm.at[srcR.at[slot, j]], gbuf.at[p], gsems[p])
        pltpu.async_copy(ed_hbm.at[dstR.at[slot, j]], ebuf.at[p], esems[p])

    def _gwait(p):
        pltpu.make_async_copy(h_hbm.at[pl.ds(0, B), :],
                              gbuf.at[p], gsems[p]).wait()
        pltpu.make_async_copy(ed_hbm.at[pl.ds(0, B), :],
                              ebuf.at[p], esems[p]).wait()

    def _sissue(slot, j, p):
        pltpu.async_copy(sbuf.at[p], acc_sh.at[dstR.at[slot, j]],
                         ssems[p], add=True)

    def _swait(p):
        pltpu.make_async_copy(sbuf.at[p], acc_sh.at[pl.ds(0, B), :],
                              ssems[p]).wait()

    _ring_issue(0)
    pltpu.sync_copy(m_hbm, m_v)
    mval = m_v[pl.ds(0, 16)][0]

    def _zrow(r, carry):
        for p in range(2):
            for ch in range(W // 16):
                sbuf[p, r, pl.ds(ch * 16, 16)] = zero16
        return carry

    lax.fori_loop(0, B, _zrow, 0)
    base = s * NPT
    nfull = NPT // B
    for j in range(nfull):
        pltpu.async_copy(sbuf.at[0, pl.ds(0, B), :],
                         acc_sh.at[pl.ds(base + j * B, B), :], ssem0)
    rem = NPT - nfull * B
    if rem:
        pltpu.async_copy(sbuf.at[0, pl.ds(0, rem), :],
                         acc_sh.at[pl.ds(base + nfull * B, rem), :], ssem0)
    for j in range(nfull):
        pltpu.make_async_copy(sbuf.at[0, pl.ds(0, B), :],
                              acc_sh.at[pl.ds(base, B), :], ssem0).wait()
    if rem:
        pltpu.make_async_copy(sbuf.at[0, pl.ds(0, rem), :],
                              acc_sh.at[pl.ds(base, rem), :], ssem0).wait()

    def _scale_batch(p):
        q16s = [q * 16 for q in range(B // 16)]
        if B % 16:
            q16s.append(B - 16)
        for gi, q16 in enumerate(q16s):
            rows = q16 + iota16
            esv = plsc.load_gather(gbuf, [psplat[p], rows, colD])
            edv = plsc.load_gather(ebuf, [psplat[p], rows, col0])
            t = esv + edv
            e = jnp.where(t >= 0.0, t, t * _f32(0.2))
            wv = jnp.exp(e - mval)
            lo = 0 if gi < len(q16s) - 1 or not B % 16 else 16 - (B % 16)
            for j2 in range(lo, 16):
                r = q16 + j2
                wsc = wv[j2]
                loads = [gbuf[p, r, pl.ds(ch * 16, 16)]
                         for ch in range(D // 16)]
                for ch in range(D // 16):
                    sbuf[p, r, pl.ds(ch * 16, 16)] = loads[ch] * wsc
            plsc.store_scatter(sbuf, [psplat[p], rows, colD], wv)

    _ring_wait()
    _gissue(0, 0, 0)
    _gissue(0, 1, 1)
    plsc.subcore_barrier()

    def _chunk(cb, carry):
        slot = lax.rem(cb, 2)
        nslot = 1 - slot

        @pl.when(cb + 1 < NCH)
        def _():
            _ring_issue(cb + 1)

        def _pair(jj, c2):
            @pl.when(jnp.logical_and(jj == 1, cb + 1 < NCH))
            def _():
                _ring_wait()

            for p in range(2):
                j = 2 * jj + p
                g = cb * CH + j
                _gwait(p)

                @pl.when(g >= 2)
                def _():
                    _swait(p)

                _scale_batch(p)
                _sissue(slot, j, p)

                @pl.when(jnp.logical_and(j + 2 < CH, g + 2 < NB))
                def _():
                    _gissue(slot, j + 2, p)

                @pl.when(jnp.logical_and(j + 2 >= CH, g + 2 < NB))
                def _():
                    _gissue(nslot, j + 2 - CH, p)
            return c2

        lax.fori_loop(0, CH // 2, _pair, 0)
        return carry

    lax.fori_loop(0, NCH, _chunk, 0)
    _swait(0)
    _swait(1)
    plsc.subcore_barrier()

    pltpu.sync_copy(acc_sh.at[pl.ds(base, NPT), :],
                    acc_out.at[c, pl.ds(base, NPT), :])


_sc_agg = pl.kernel(
    _sc_agg_body,
    out_type=jax.ShapeDtypeStruct((NC, N, W), _f32),
    mesh=plsc.VectorSubcoreMesh(core_axis_name="c", subcore_axis_name="s"),
    compiler_params=_SC_PARAMS,
    scratch_types=(
        [pltpu.VMEM((2, CH, B), _i32),
         pltpu.VMEM((2, CH, B), _i32),
         pltpu.VMEM((2, B, W), _f32),
         pltpu.VMEM((2, B, 16), _f32),
         pltpu.VMEM((2, B, W), _f32),
         pltpu.VMEM((16,), _f32)]
        + [pltpu.VMEM_SHARED((N, W), _f32)]
        + [pltpu.SemaphoreType.DMA for _ in range(7)]
    ),
)


def _dense_tail(h, asrc_ref, adst_ref, hp_ref, ed_ref, m_ref):
    es = jnp.dot(h, asrc_ref[...], preferred_element_type=_f32)
    ed = jnp.dot(h, adst_ref[...], preferred_element_type=_f32)
    hp_ref[...] = jnp.concatenate(
        [h, es, jnp.zeros((N, W - D - 1), _f32)], axis=1)
    ed_ref[...] = jnp.concatenate([ed, jnp.zeros((N, 15), _f32)], axis=1)
    mm = jnp.max(es) + jnp.max(ed)
    mb = jnp.where(mm >= 0.0, mm, mm * 0.2)
    m_ref[...] = jnp.full((1, 16), mb, _f32)


def _tc1_body(x_ref, w_ref, asrc_ref, adst_ref, hp_ref, ed_ref, m_ref):
    h = jnp.dot(x_ref[...], w_ref[...], preferred_element_type=_f32)
    _dense_tail(h, asrc_ref, adst_ref, hp_ref, ed_ref, m_ref)


def _combine(acc_ref, b_ref):
    a = acc_ref[0, :, :D] + acc_ref[1, :, :D]
    den = acc_ref[0, :, D:D + 1] + acc_ref[1, :, D:D + 1]
    den = jnp.where(den == 0.0, _f32(1.0), den)
    return jnp.maximum(a / den + b_ref[...], 0.0)


def _tc2_body(acc_ref, b_ref, w_ref, asrc_ref, adst_ref,
              hp_ref, ed_ref, m_ref):
    emb = _combine(acc_ref, b_ref)
    h = jnp.dot(emb, w_ref[...], preferred_element_type=_f32)
    _dense_tail(h, asrc_ref, adst_ref, hp_ref, ed_ref, m_ref)


def _tc3_body(acc_ref, b_ref, wo_ref, bo_ref, o_ref):
    emb = _combine(acc_ref, b_ref)
    o_ref[...] = jnp.dot(emb, wo_ref[...], preferred_element_type=_f32) + bo_ref[...]


_dense_out = (jax.ShapeDtypeStruct((N, W), _f32),
              jax.ShapeDtypeStruct((N, 16), _f32),
              jax.ShapeDtypeStruct((1, 16), _f32))

_tc1 = pl.pallas_call(_tc1_body, out_shape=_dense_out)
_tc2 = pl.pallas_call(_tc2_body, out_shape=_dense_out)
_tc3 = pl.pallas_call(_tc3_body, out_shape=jax.ShapeDtypeStruct((N, 1), _f32))


def kernel(x, edge_index, W1, a1_src, a1_dst, b1, W2, a2_src, a2_dst, b2, Wo, bo):
    src3 = edge_index[0].reshape(NW, NB, B)
    dst3 = edge_index[1].reshape(NW, NB, B)

    hp1, ed1, m1 = _tc1(x, W1, a1_src.reshape(D, 1), a1_dst.reshape(D, 1))
    acc1 = _sc_agg(hp1, ed1, m1.reshape(16), src3, dst3)
    hp2, ed2, m2 = _tc2(acc1, b1.reshape(1, D), W2,
                        a2_src.reshape(D, 1), a2_dst.reshape(D, 1))
    acc2 = _sc_agg(hp2, ed2, m2.reshape(16), src3, dst3)
    return _tc3(acc2, b2.reshape(1, D), Wo, bo.reshape(1, 1))
